# Initial kernel scaffold; baseline (speedup 1.0000x reference)
import jax
import jax.numpy as jnp
from jax.experimental import pallas as pl


def kernel(x, edge_index, W):
    raise NotImplementedError("write your pallas kernel here")


# speedup vs baseline: 12.2357x; 12.2357x over previous
import functools

import jax
import jax.numpy as jnp
from jax import lax
from jax.experimental import pallas as pl
from jax.experimental.pallas import tpu as pltpu
from jax.experimental.pallas import tpu_sc as plsc

N = 10000
E = 160000
FEAT = 256
EMB = 256
HALF = EMB // 2

NC, NS, L = 2, 16, 16
CH = 128
NCHUNK = E // CH
ITERS = -(-NCHUNK // NS)
NP = 10240
DEG_ROWS = NP // NS
ACC_ROWS = NP // NS

_mesh = functools.partial(
    plsc.VectorSubcoreMesh, core_axis_name="c", subcore_axis_name="s"
)


def _deg_body(dst_hbm, deg_hbm, ones_v, dst_v, deg_sh):
    c = lax.axis_index("c")
    s = lax.axis_index("s")

    @pl.when(c == 0)
    def _():
        for i in range(CH // L):
            ones_v[pl.ds(i * L, L)] = jnp.ones((L,), jnp.float32)
        base = s * DEG_ROWS
        for j in range(DEG_ROWS // CH):
            pltpu.sync_copy(ones_v, deg_sh.at[pl.ds(base + j * CH, CH)])
        plsc.subcore_barrier()

        def loop(k, carry):
            chunk = k * NS + s

            @pl.when(chunk < NCHUNK)
            def _():
                pltpu.sync_copy(dst_hbm.at[pl.ds(chunk * CH, CH)], dst_v)
                pltpu.sync_copy(ones_v, deg_sh.at[dst_v], add=True)

            return carry

        lax.fori_loop(0, ITERS, loop, 0)
        plsc.subcore_barrier()
        pltpu.sync_copy(
            deg_sh.at[pl.ds(base, DEG_ROWS)], deg_hbm.at[pl.ds(base, DEG_ROWS)]
        )


def _degree(dst):
    run = pl.kernel(
        _deg_body,
        out_type=jax.ShapeDtypeStruct((NP,), jnp.float32),
        mesh=_mesh(),
        scratch_types=[
            pltpu.VMEM((CH,), jnp.float32),
            pltpu.VMEM((CH,), jnp.int32),
            pltpu.VMEM_SHARED((NP,), jnp.float32),
        ],
    )
    return run(dst)


def _gemm_body(x_ref, w_ref, deg_ref, h0_ref, h1_ref):
    h = lax.dot_general(
        x_ref[...], w_ref[...], (((1,), (1,)), ((), ())),
        preferred_element_type=jnp.float32,
    )
    h = h * lax.rsqrt(deg_ref[...])
    h0_ref[...] = h[:, :HALF]
    h1_ref[...] = h[:, HALF:]


def _gemm_scaled(x, W, deg):
    R = 1024
    return pl.pallas_call(
        _gemm_body,
        grid=(NP // R,),
        in_specs=[
            pl.BlockSpec((R, FEAT), lambda i: (i, 0)),
            pl.BlockSpec((EMB, FEAT), lambda i: (0, 0)),
            pl.BlockSpec((R, 1), lambda i: (i, 0)),
        ],
        out_specs=[
            pl.BlockSpec((R, HALF), lambda i: (i, 0)),
            pl.BlockSpec((R, HALF), lambda i: (i, 0)),
        ],
        out_shape=[jax.ShapeDtypeStruct((NP, HALF), jnp.float32)] * 2,
    )(x, W, deg)


def _spmm_body(src_hbm, dst_hbm, h0_hbm, h1_hbm, a0_hbm, a1_hbm,
               idx_v, dst_v, rows_v, acc_sh, sem):
    c = lax.axis_index("c")
    s = lax.axis_index("s")
    base = s * ACC_ROWS

    def half(h_hbm, out_hbm):
        pltpu.sync_copy(
            h_hbm.at[pl.ds(base, ACC_ROWS)], acc_sh.at[pl.ds(base, ACC_ROWS)]
        )
        plsc.subcore_barrier()

        def loop(k, carry):
            chunk = k * NS + s

            @pl.when(chunk < NCHUNK)
            def _():
                off = chunk * CH
                pltpu.sync_copy(src_hbm.at[pl.ds(off, CH)], idx_v)
                pltpu.sync_copy(dst_hbm.at[pl.ds(off, CH)], dst_v)
                pltpu.async_copy(h_hbm.at[idx_v], rows_v, sem).wait()
                pltpu.sync_copy(rows_v, acc_sh.at[dst_v], add=True)

            return carry

        lax.fori_loop(0, ITERS, loop, 0)
        plsc.subcore_barrier()
        pltpu.sync_copy(
            acc_sh.at[pl.ds(base, ACC_ROWS)], out_hbm.at[pl.ds(base, ACC_ROWS)]
        )

    @pl.when(c == 0)
    def _():
        half(h0_hbm, a0_hbm)

    @pl.when(c == 1)
    def _():
        half(h1_hbm, a1_hbm)


def _spmm(src, dst, h0, h1):
    run = pl.kernel(
        _spmm_body,
        out_type=[jax.ShapeDtypeStruct((NP, HALF), jnp.float32)] * 2,
        mesh=_mesh(),
        scratch_types=[
            pltpu.VMEM((CH,), jnp.int32),
            pltpu.VMEM((CH,), jnp.int32),
            pltpu.VMEM((CH, HALF), jnp.float32),
            pltpu.VMEM_SHARED((NP, HALF), jnp.float32),
            pltpu.SemaphoreType.DMA,
        ],
    )
    return run(src, dst, h0, h1)


def _combine_body(a0_ref, a1_ref, deg_ref, out_ref):
    isd = lax.rsqrt(deg_ref[...])
    out_ref[:, :HALF] = a0_ref[...] * isd
    out_ref[:, HALF:] = a1_ref[...] * isd


def _combine(a0, a1, deg):
    R = 1000
    return pl.pallas_call(
        _combine_body,
        grid=(N // R,),
        in_specs=[
            pl.BlockSpec((R, HALF), lambda i: (i, 0)),
            pl.BlockSpec((R, HALF), lambda i: (i, 0)),
            pl.BlockSpec((R, 1), lambda i: (i, 0)),
        ],
        out_specs=pl.BlockSpec((R, EMB), lambda i: (i, 0)),
        out_shape=jax.ShapeDtypeStruct((N, EMB), jnp.float32),
    )(a0, a1, deg)


def kernel(x, edge_index, W):
    src = edge_index[0].astype(jnp.int32)
    dst = edge_index[1].astype(jnp.int32)
    deg = _degree(dst).reshape(NP, 1)
    h0, h1 = _gemm_scaled(x, W, deg)
    a0, a1 = _spmm(src, dst, h0, h1)
    return _combine(a0, a1, deg)
